# baseline (speedup 1.0000x reference)
import functools

import jax
import jax.numpy as jnp
from jax import lax
from jax.experimental import pallas as pl
from jax.experimental.pallas import tpu as pltpu
from jax.experimental.pallas import tpu_sc as plsc

_NUM_ITEMS = 1_000_000
_TBL_PAD = 1_048_576
_BATCH = 16_384
_HIST = 200
_N = _BATCH * _HIST

_NC = 2
_NS = 16
_NW = _NC * _NS

_SUB = 128
_GROUP = 2048
_ROWS_PER_GROUP = _GROUP // _SUB
_PER_W = _N // _NW
_N_GROUPS = _PER_W // _GROUP


def _tc_sigmoid_body(x_ref, o_ref):
    x = x_ref[...]
    o_ref[...] = 1.0 / (1.0 + jnp.exp(-4.0 * x))


def _tc_sigmoid(tbl2d):
    return pl.pallas_call(
        _tc_sigmoid_body,
        out_shape=jax.ShapeDtypeStruct(tbl2d.shape, jnp.float32),
    )(tbl2d)


def _sc_gather_body(tbl_hbm, idx_hbm, out_hbm, shared_tbl, idx_v, rows_v,
                    si0, si1, sg0, sg1, so0, so1):
    sid = lax.axis_index("s")
    wid = sid * _NC + lax.axis_index("c")
    base_row = wid * (_PER_W // _SUB)
    si = (si0, si1)
    sg = (sg0, sg1)
    so = (so0, so1)

    def idx_copy(g, b, sem):
        return pltpu.make_async_copy(
            idx_hbm.at[pl.ds(base_row + g * _ROWS_PER_GROUP, _ROWS_PER_GROUP), :],
            idx_v.at[b],
            sem,
        )

    def out_copy(g, b, sem):
        return pltpu.make_async_copy(
            rows_v.at[b],
            out_hbm.at[pl.ds((base_row + g * _ROWS_PER_GROUP) * _SUB, _GROUP)],
            sem,
        )

    idx_copy(0, 0, si[0]).start()
    idx_copy(1, 1, si[1]).start()

    pltpu.sync_copy(
        tbl_hbm.at[pl.ds(sid * (_TBL_PAD // _NS), _TBL_PAD // _NS)],
        shared_tbl.at[pl.ds(sid * (_TBL_PAD // _NS), _TBL_PAD // _NS)],
    )

    plsc.subcore_barrier()

    n_iter = _N_GROUPS // 2

    def body(i, carry):
        for b in range(2):
            g = 2 * i + b
            idx_copy(g, b, si[b]).wait()

            @pl.when(g >= 2)
            def _():
                out_copy(g - 2, b, so[b]).wait()

            cps = []
            for j in range(_ROWS_PER_GROUP):
                cps.append(
                    pltpu.async_copy(
                        shared_tbl.at[idx_v.at[b].at[j]],
                        rows_v.at[b].at[pl.ds(j * _SUB, _SUB)],
                        sg[b],
                    )
                )

            for cp in cps:
                cp.wait()

            @pl.when(i < n_iter - 1)
            def _():
                idx_copy(g + 2, b, si[b]).start()

            out_copy(g, b, so[b]).start()
        return carry

    lax.fori_loop(0, n_iter, body, 0)
    out_copy(_N_GROUPS - 2, 0, so[0]).wait()
    out_copy(_N_GROUPS - 1, 1, so[1]).wait()


@functools.partial(
    pl.kernel,
    mesh=plsc.VectorSubcoreMesh(core_axis_name="c", subcore_axis_name="s"),
    out_type=jax.ShapeDtypeStruct((_N,), jnp.float32),
    scratch_types=[
        pltpu.VMEM_SHARED((_TBL_PAD,), jnp.float32),
        pltpu.VMEM((2, _ROWS_PER_GROUP, _SUB), jnp.int32),
        pltpu.VMEM((2, _GROUP), jnp.float32),
        pltpu.SemaphoreType.DMA,
        pltpu.SemaphoreType.DMA,
        pltpu.SemaphoreType.DMA,
        pltpu.SemaphoreType.DMA,
        pltpu.SemaphoreType.DMA,
        pltpu.SemaphoreType.DMA,
    ],
)
def _sc_gather(tbl_hbm, idx_hbm, out_hbm, shared_tbl, idx_v, rows_v,
               si0, si1, sg0, sg1, so0, so1):
    _sc_gather_body(tbl_hbm, idx_hbm, out_hbm, shared_tbl, idx_v, rows_v,
                    si0, si1, sg0, sg1, so0, so1)


def kernel(inputs, item_rating_logits):
    b, h = inputs.shape[1], inputs.shape[2]
    idx2d = inputs.reshape(b * h // _SUB, _SUB)
    padded = jnp.pad(item_rating_logits, (0, _TBL_PAD - _NUM_ITEMS))
    tbl2d = padded.reshape(1024, 1024)
    ratings = _tc_sigmoid(tbl2d).reshape(-1)
    out = _sc_gather(ratings, idx2d)
    return out.reshape(b, h)


# speedup vs baseline: 1.0015x; 1.0015x over previous
import functools

import jax
import jax.numpy as jnp
from jax import lax
from jax.experimental import pallas as pl
from jax.experimental.pallas import tpu as pltpu
from jax.experimental.pallas import tpu_sc as plsc

_NUM_ITEMS = 1_000_000
_TBL_PAD = 1_048_576
_BATCH = 16_384
_HIST = 200
_N = _BATCH * _HIST

_NC = 2
_NS = 16
_NW = _NC * _NS

_SUB = 128
_GROUP = 2048
_ROWS_PER_GROUP = _GROUP // _SUB
_PER_W = _N // _NW
_N_GROUPS = _PER_W // _GROUP
_LROWS = _N // _SUB


def _tc_sigmoid_body(x_ref, o_ref):
    x = x_ref[...]
    o_ref[...] = 1.0 / (1.0 + jnp.exp(-4.0 * x))


def _tc_sigmoid(tbl2d):
    return pl.pallas_call(
        _tc_sigmoid_body,
        out_shape=jax.ShapeDtypeStruct(tbl2d.shape, jnp.float32),
    )(tbl2d)


def _sc_gather_body(tbl_hbm, idx_hbm, out_hbm, shared_tbl, idx_v, rows_v,
                    si0, si1, sg0, sg1, so0, so1):
    sid = lax.axis_index("s")
    wid = sid * _NC + lax.axis_index("c")
    base_row = wid * (_PER_W // _SUB)
    si = (si0, si1)
    sg = (sg0, sg1)
    so = (so0, so1)

    def idx_copy(g, b, sem):
        return pltpu.make_async_copy(
            idx_hbm.at[pl.ds(base_row + g * _ROWS_PER_GROUP, _ROWS_PER_GROUP), :],
            idx_v.at[b],
            sem,
        )

    def out_copy(g, b, sem):
        return pltpu.make_async_copy(
            rows_v.at[b],
            out_hbm.at[pl.ds(base_row + g * _ROWS_PER_GROUP, _ROWS_PER_GROUP), :],
            sem,
        )

    idx_copy(0, 0, si[0]).start()
    idx_copy(1, 1, si[1]).start()

    pltpu.sync_copy(
        tbl_hbm.at[pl.ds(sid * (_TBL_PAD // _NS), _TBL_PAD // _NS)],
        shared_tbl.at[pl.ds(sid * (_TBL_PAD // _NS), _TBL_PAD // _NS)],
    )

    plsc.subcore_barrier()

    n_iter = _N_GROUPS // 2

    def body(i, carry):
        for b in range(2):
            g = 2 * i + b
            idx_copy(g, b, si[b]).wait()

            @pl.when(g >= 2)
            def _():
                out_copy(g - 2, b, so[b]).wait()

            cps = []
            for j in range(_ROWS_PER_GROUP):
                cps.append(
                    pltpu.async_copy(
                        shared_tbl.at[idx_v.at[b].at[j]],
                        rows_v.at[b].at[j],
                        sg[b],
                    )
                )
            for cp in cps:
                cp.wait()

            @pl.when(i < n_iter - 1)
            def _():
                idx_copy(g + 2, b, si[b]).start()

            out_copy(g, b, so[b]).start()
        return carry

    lax.fori_loop(0, n_iter, body, 0)
    out_copy(_N_GROUPS - 2, 0, so[0]).wait()
    out_copy(_N_GROUPS - 1, 1, so[1]).wait()


@functools.partial(
    pl.kernel,
    mesh=plsc.VectorSubcoreMesh(core_axis_name="c", subcore_axis_name="s"),
    out_type=jax.ShapeDtypeStruct((_LROWS, _SUB), jnp.float32),
    scratch_types=[
        pltpu.VMEM_SHARED((_TBL_PAD,), jnp.float32),
        pltpu.VMEM((2, _ROWS_PER_GROUP, _SUB), jnp.int32),
        pltpu.VMEM((2, _ROWS_PER_GROUP, _SUB), jnp.float32),
        pltpu.SemaphoreType.DMA,
        pltpu.SemaphoreType.DMA,
        pltpu.SemaphoreType.DMA,
        pltpu.SemaphoreType.DMA,
        pltpu.SemaphoreType.DMA,
        pltpu.SemaphoreType.DMA,
    ],
)
def _sc_gather(tbl_hbm, idx_hbm, out_hbm, shared_tbl, idx_v, rows_v,
               si0, si1, sg0, sg1, so0, so1):
    _sc_gather_body(tbl_hbm, idx_hbm, out_hbm, shared_tbl, idx_v, rows_v,
                    si0, si1, sg0, sg1, so0, so1)


def kernel(inputs, item_rating_logits):
    b, h = inputs.shape[1], inputs.shape[2]
    idx2d = inputs.reshape(b * h // _SUB, _SUB)
    padded = jnp.pad(item_rating_logits, (0, _TBL_PAD - _NUM_ITEMS))
    tbl2d = padded.reshape(1024, 1024)
    ratings = _tc_sigmoid(tbl2d).reshape(-1)
    out2d = _sc_gather(ratings, idx2d)
    return out2d.reshape(b, h)


# speedup vs baseline: 1.4298x; 1.4276x over previous
import functools

import jax
import jax.numpy as jnp
from jax import lax
from jax.experimental import pallas as pl
from jax.experimental.pallas import tpu as pltpu
from jax.experimental.pallas import tpu_sc as plsc

_NUM_ITEMS = 1_000_000
_TBL_PAD = 1_048_576
_BATCH = 16_384
_HIST = 200
_N = _BATCH * _HIST

_NC = 2
_NS = 16
_NW = _NC * _NS

_ROWS_PER_W = _BATCH // _NW
_GROUP_ROWS = 8
_N_GROUPS = _ROWS_PER_W // _GROUP_ROWS
_SPLIT = ((0, 128), (128, _HIST - 128))


def _tc_sigmoid_body(x_ref, o_ref):
    x = x_ref[...]
    o_ref[...] = 1.0 / (1.0 + jnp.exp(-4.0 * x))


def _tc_sigmoid(tbl2d):
    return pl.pallas_call(
        _tc_sigmoid_body,
        out_shape=jax.ShapeDtypeStruct(tbl2d.shape, jnp.float32),
    )(tbl2d)


def _sc_gather_body(tbl_hbm, idx_hbm, out_hbm, shared_tbl, idx_v, rows_v,
                    si0, si1, sg0, sg1, so0, so1):
    sid = lax.axis_index("s")
    wid = sid * _NC + lax.axis_index("c")
    row_base = wid * _ROWS_PER_W
    si = (si0, si1)
    sg = (sg0, sg1)
    so = (so0, so1)

    def idx_copy(g, b, sem):
        return pltpu.make_async_copy(
            idx_hbm.at[pl.ds(row_base + g * _GROUP_ROWS, _GROUP_ROWS), :],
            idx_v.at[b],
            sem,
        )

    def out_copy(g, b, sem):
        return pltpu.make_async_copy(
            rows_v.at[b],
            out_hbm.at[pl.ds(row_base + g * _GROUP_ROWS, _GROUP_ROWS), :],
            sem,
        )

    idx_copy(0, 0, si[0]).start()
    idx_copy(1, 1, si[1]).start()

    pltpu.sync_copy(
        tbl_hbm.at[pl.ds(sid * (_TBL_PAD // _NS), _TBL_PAD // _NS)],
        shared_tbl.at[pl.ds(sid * (_TBL_PAD // _NS), _TBL_PAD // _NS)],
    )

    plsc.subcore_barrier()

    n_iter = _N_GROUPS // 2

    def body(i, carry):
        for b in range(2):
            g = 2 * i + b
            idx_copy(g, b, si[b]).wait()

            @pl.when(g >= 2)
            def _():
                out_copy(g - 2, b, so[b]).wait()

            cps = []
            for r in range(_GROUP_ROWS):
                for off, cnt in _SPLIT:
                    cps.append(
                        pltpu.async_copy(
                            shared_tbl.at[idx_v.at[b].at[r, pl.ds(off, cnt)]],
                            rows_v.at[b].at[r, pl.ds(off, cnt)],
                            sg[b],
                        )
                    )
            for cp in cps:
                cp.wait()

            @pl.when(i < n_iter - 1)
            def _():
                idx_copy(g + 2, b, si[b]).start()

            out_copy(g, b, so[b]).start()
        return carry

    lax.fori_loop(0, n_iter, body, 0)
    out_copy(_N_GROUPS - 2, 0, so[0]).wait()
    out_copy(_N_GROUPS - 1, 1, so[1]).wait()


@functools.partial(
    pl.kernel,
    mesh=plsc.VectorSubcoreMesh(core_axis_name="c", subcore_axis_name="s"),
    out_type=jax.ShapeDtypeStruct((_BATCH, _HIST), jnp.float32),
    scratch_types=[
        pltpu.VMEM_SHARED((_TBL_PAD,), jnp.float32),
        pltpu.VMEM((2, _GROUP_ROWS, _HIST), jnp.int32),
        pltpu.VMEM((2, _GROUP_ROWS, _HIST), jnp.float32),
        pltpu.SemaphoreType.DMA,
        pltpu.SemaphoreType.DMA,
        pltpu.SemaphoreType.DMA,
        pltpu.SemaphoreType.DMA,
        pltpu.SemaphoreType.DMA,
        pltpu.SemaphoreType.DMA,
    ],
)
def _sc_gather(tbl_hbm, idx_hbm, out_hbm, shared_tbl, idx_v, rows_v,
               si0, si1, sg0, sg1, so0, so1):
    _sc_gather_body(tbl_hbm, idx_hbm, out_hbm, shared_tbl, idx_v, rows_v,
                    si0, si1, sg0, sg1, so0, so1)


def kernel(inputs, item_rating_logits):
    b, h = inputs.shape[1], inputs.shape[2]
    idx2d = inputs.reshape(b, h)
    padded = jnp.pad(item_rating_logits, (0, _TBL_PAD - _NUM_ITEMS))
    tbl2d = padded.reshape(1024, 1024)
    ratings = _tc_sigmoid(tbl2d).reshape(-1)
    return _sc_gather(ratings, idx2d)


# speedup vs baseline: 1.4427x; 1.0091x over previous
import functools

import jax
import jax.numpy as jnp
from jax import lax
from jax.experimental import pallas as pl
from jax.experimental.pallas import tpu as pltpu
from jax.experimental.pallas import tpu_sc as plsc

_NUM_ITEMS = 1_000_000
_TBL_PAD = 1_048_576
_BATCH = 16_384
_HIST = 200
_N = _BATCH * _HIST

_NC = 2
_NS = 16
_NW = _NC * _NS

_ROWS_PER_W = _BATCH // _NW
_GROUP_ROWS = 8
_N_GROUPS = _ROWS_PER_W // _GROUP_ROWS
_SPLIT = ((0, 128), (128, _HIST - 128))


def _tc_sigmoid_body(x_ref, o_ref):
    x = x_ref[...]
    o_ref[...] = 1.0 / (1.0 + jnp.exp(-4.0 * x))


def _tc_sigmoid(tbl2d):
    return pl.pallas_call(
        _tc_sigmoid_body,
        out_shape=jax.ShapeDtypeStruct(tbl2d.shape, jnp.float32),
        in_specs=[pl.BlockSpec((1024, 128), lambda i: (i, 0))],
        out_specs=pl.BlockSpec((1024, 128), lambda i: (i, 0)),
        grid=(tbl2d.shape[0] // 1024,),
    )(tbl2d)


def _sc_gather_body(tbl_hbm, idx_hbm, out_hbm, shared_tbl, idx_v, rows_v,
                    si0, si1, sg0, sg1, so0, so1):
    sid = lax.axis_index("s")
    wid = sid * _NC + lax.axis_index("c")
    row_base = wid * _ROWS_PER_W
    si = (si0, si1)
    sg = (sg0, sg1)
    so = (so0, so1)

    def idx_copy(g, b, sem):
        return pltpu.make_async_copy(
            idx_hbm.at[pl.ds(row_base + g * _GROUP_ROWS, _GROUP_ROWS), :],
            idx_v.at[b],
            sem,
        )

    def out_copy(g, b, sem):
        return pltpu.make_async_copy(
            rows_v.at[b],
            out_hbm.at[pl.ds(row_base + g * _GROUP_ROWS, _GROUP_ROWS), :],
            sem,
        )

    idx_copy(0, 0, si[0]).start()
    idx_copy(1, 1, si[1]).start()

    pltpu.sync_copy(
        tbl_hbm.at[pl.ds(sid * (_TBL_PAD // _NS), _TBL_PAD // _NS)],
        shared_tbl.at[pl.ds(sid * (_TBL_PAD // _NS), _TBL_PAD // _NS)],
    )

    plsc.subcore_barrier()

    n_iter = _N_GROUPS // 2

    def body(i, carry):
        for b in range(2):
            g = 2 * i + b
            idx_copy(g, b, si[b]).wait()

            @pl.when(g >= 2)
            def _():
                out_copy(g - 2, b, so[b]).wait()

            cps = []
            for r in range(_GROUP_ROWS):
                for off, cnt in _SPLIT:
                    cps.append(
                        pltpu.async_copy(
                            shared_tbl.at[idx_v.at[b].at[r, pl.ds(off, cnt)]],
                            rows_v.at[b].at[r, pl.ds(off, cnt)],
                            sg[b],
                        )
                    )
            for cp in cps:
                cp.wait()

            @pl.when(i < n_iter - 1)
            def _():
                idx_copy(g + 2, b, si[b]).start()

            out_copy(g, b, so[b]).start()
        return carry

    lax.fori_loop(0, n_iter, body, 0)
    out_copy(_N_GROUPS - 2, 0, so[0]).wait()
    out_copy(_N_GROUPS - 1, 1, so[1]).wait()


@functools.partial(
    pl.kernel,
    mesh=plsc.VectorSubcoreMesh(core_axis_name="c", subcore_axis_name="s"),
    out_type=jax.ShapeDtypeStruct((_BATCH, _HIST), jnp.float32),
    scratch_types=[
        pltpu.VMEM_SHARED((_TBL_PAD,), jnp.float32),
        pltpu.VMEM((2, _GROUP_ROWS, _HIST), jnp.int32),
        pltpu.VMEM((2, _GROUP_ROWS, _HIST), jnp.float32),
        pltpu.SemaphoreType.DMA,
        pltpu.SemaphoreType.DMA,
        pltpu.SemaphoreType.DMA,
        pltpu.SemaphoreType.DMA,
        pltpu.SemaphoreType.DMA,
        pltpu.SemaphoreType.DMA,
    ],
)
def _sc_gather(tbl_hbm, idx_hbm, out_hbm, shared_tbl, idx_v, rows_v,
               si0, si1, sg0, sg1, so0, so1):
    _sc_gather_body(tbl_hbm, idx_hbm, out_hbm, shared_tbl, idx_v, rows_v,
                    si0, si1, sg0, sg1, so0, so1)


def kernel(inputs, item_rating_logits):
    b, h = inputs.shape[1], inputs.shape[2]
    idx2d = inputs.reshape(b, h)
    padded = jnp.pad(item_rating_logits, (0, _TBL_PAD - _NUM_ITEMS))
    tbl2d = padded.reshape(8192, 128)
    ratings = _tc_sigmoid(tbl2d).reshape(-1)
    return _sc_gather(ratings, idx2d)
